# baseline (speedup 1.0000x reference)
import functools

import jax
import jax.numpy as jnp
from jax import lax
from jax.experimental import pallas as pl
from jax.experimental.pallas import tpu as pltpu
from jax.experimental.pallas import tpu_sc as plsc

_NC, _NS, _L = 2, 16, 16
_NW = _NC * _NS
_CHUNK = 8
_UNROLL = 8


@functools.cache
def _build_sc_kernel(B, S, D, dtype):
    rows_per_w = S // _NW
    n_chunks = rows_per_w // _CHUNK
    cw = _CHUNK * D

    mesh = plsc.VectorSubcoreMesh(core_axis_name="c", subcore_axis_name="s")

    @functools.partial(
        pl.kernel,
        out_type=jax.ShapeDtypeStruct((B, S * D), dtype),
        mesh=mesh,
        scratch_types=[
            pltpu.VMEM((2, cw), jnp.float32),
            pltpu.VMEM((2, B, cw), jnp.float32),
            pltpu.SemaphoreType.DMA,
            pltpu.SemaphoreType.DMA,
            pltpu.SemaphoreType.DMA,
            pltpu.SemaphoreType.DMA,
        ],
    )
    def sc_kernel(in_hbm, emb_hbm, out_hbm, emb_v, io_v,
                  lsem0, lsem1, ssem0, ssem1):
        wid = lax.axis_index("s") * _NC + lax.axis_index("c")
        base = wid * rows_per_w * D
        lsems = (lsem0, lsem1)
        ssems = (ssem0, ssem1)

        def start_loads(ci, slot):
            off = base + ci * cw
            hs = [pltpu.async_copy(emb_hbm.at[pl.ds(off, cw)],
                                   emb_v.at[slot], lsems[slot])]
            for b in range(B):
                hs.append(pltpu.async_copy(in_hbm.at[b, pl.ds(off, cw)],
                                           io_v.at[slot, b], lsems[slot]))
            return hs

        def start_stores(ci, slot):
            off = base + ci * cw
            return [pltpu.async_copy(io_v.at[slot, b],
                                     out_hbm.at[b, pl.ds(off, cw)], ssems[slot])
                    for b in range(B)]

        def compute(slot):
            @pl.loop(0, cw // _L, unroll=_UNROLL)
            def _vec(i):
                sl = pl.ds(i * _L, _L)
                e = emb_v[slot, sl]
                for b in range(B):
                    plsc.addupdate(io_v.at[slot, b, sl], e)

        pending_loads = {0: start_loads(0, 0)}
        pending_stores = {}
        for ci in range(n_chunks):
            slot = ci % 2
            if ci + 1 < n_chunks:
                if ci - 1 in pending_stores:
                    for h in pending_stores.pop(ci - 1):
                        h.wait()
                pending_loads[ci + 1] = start_loads(ci + 1, (ci + 1) % 2)
            for h in pending_loads.pop(ci):
                h.wait()
            compute(slot)
            pending_stores[ci] = start_stores(ci, slot)
        for hs in pending_stores.values():
            for h in hs:
                h.wait()

    def run(inputs, pos):
        out = sc_kernel(inputs.reshape(B, S * D), pos.reshape(S * D))
        return out.reshape(B, S, D)

    return run


def kernel(inputs, embeddings):
    B, S, D = inputs.shape
    pos = embeddings[:S]
    return _build_sc_kernel(B, S, D, inputs.dtype)(inputs, pos)


# speedup vs baseline: 1.0129x; 1.0129x over previous
import functools

import jax
import jax.numpy as jnp
from jax import lax
from jax.experimental import pallas as pl
from jax.experimental.pallas import tpu as pltpu
from jax.experimental.pallas import tpu_sc as plsc

_NC, _NS, _L = 2, 16, 16
_NW = _NC * _NS
_CHUNK = 8
_NSLOT = 3
_UNROLL = 8


@functools.cache
def _build_sc_kernel(B, S, D, dtype):
    rows_per_w = S // _NW
    n_chunks = rows_per_w // _CHUNK
    cw = _CHUNK * D

    mesh = plsc.VectorSubcoreMesh(core_axis_name="c", subcore_axis_name="s")

    @functools.partial(
        pl.kernel,
        out_type=jax.ShapeDtypeStruct((B, S * D), dtype),
        mesh=mesh,
        scratch_types=[
            pltpu.VMEM((_NSLOT, cw), jnp.float32),
            pltpu.VMEM((_NSLOT, B, cw), jnp.float32),
            pltpu.SemaphoreType.DMA((_NSLOT,)),
            pltpu.SemaphoreType.DMA((_NSLOT,)),
        ],
    )
    def sc_kernel(in_hbm, emb_hbm, out_hbm, emb_v, io_v, lsem, ssem):
        wid = lax.axis_index("s") * _NC + lax.axis_index("c")
        base = wid * rows_per_w * D

        def load_descs(ci, slot):
            off = base + ci * cw
            ds = [pltpu.make_async_copy(emb_hbm.at[pl.ds(off, cw)],
                                        emb_v.at[slot], lsem.at[slot])]
            for b in range(B):
                ds.append(pltpu.make_async_copy(in_hbm.at[b, pl.ds(off, cw)],
                                                io_v.at[slot, b], lsem.at[slot]))
            return ds

        def store_descs(ci, slot):
            off = base + ci * cw
            return [pltpu.make_async_copy(io_v.at[slot, b],
                                          out_hbm.at[b, pl.ds(off, cw)],
                                          ssem.at[slot])
                    for b in range(B)]

        def start_loads(ci, slot):
            for d in load_descs(ci, slot):
                d.start()

        def wait_loads(ci, slot):
            for d in load_descs(ci, slot):
                d.wait()

        def start_stores(ci, slot):
            for d in store_descs(ci, slot):
                d.start()

        def wait_stores(ci, slot):
            for d in store_descs(ci, slot):
                d.wait()

        start_loads(0, 0)

        @pl.loop(0, n_chunks)
        def _chunk(ci):
            slot = lax.rem(ci, _NSLOT)
            nxt = lax.rem(ci + 1, _NSLOT)

            @pl.when(ci + 1 < n_chunks)
            def _():
                @pl.when(ci >= _NSLOT - 1)
                def _():
                    wait_stores(ci - (_NSLOT - 1), nxt)
                start_loads(ci + 1, nxt)

            wait_loads(ci, slot)

            @pl.loop(0, cw // _L, unroll=_UNROLL)
            def _vec(i):
                sl = pl.ds(i * _L, _L)
                e = emb_v[slot, sl]
                for b in range(B):
                    plsc.addupdate(io_v.at[slot, b, sl], e)

            start_stores(ci, slot)

        for tail in range(n_chunks - (_NSLOT - 1), n_chunks):
            wait_stores(tail, tail % _NSLOT)

    def run(inputs, pos):
        out = sc_kernel(inputs.reshape(B, S * D), pos.reshape(S * D))
        return out.reshape(B, S, D)

    return run


def kernel(inputs, embeddings):
    B, S, D = inputs.shape
    pos = embeddings[:S]
    return _build_sc_kernel(B, S, D, inputs.dtype)(inputs, pos)


# speedup vs baseline: 3.3392x; 3.2966x over previous
import functools

import jax
import jax.numpy as jnp
from jax import lax
from jax.experimental import pallas as pl
from jax.experimental.pallas import tpu as pltpu
from jax.experimental.pallas import tpu_sc as plsc

_NC, _NS, _L = 2, 16, 16
_NW = _NC * _NS
_CHUNK = 8
_NSLOT = 3
_UNROLL = 8


@functools.cache
def _build_sc_kernel(B, S, D, dtype):
    rows_per_w = S // _NW
    n_chunks = rows_per_w // _CHUNK
    vecs_per_row = D // _L

    mesh = plsc.VectorSubcoreMesh(core_axis_name="c", subcore_axis_name="s")

    @functools.partial(
        pl.kernel,
        out_type=jax.ShapeDtypeStruct((B, S, D), dtype),
        mesh=mesh,
        scratch_types=[
            pltpu.VMEM((_NSLOT, _CHUNK, D), jnp.float32),
            pltpu.VMEM((_NSLOT, B, _CHUNK, D), jnp.float32),
            pltpu.SemaphoreType.DMA((_NSLOT,)),
            pltpu.SemaphoreType.DMA((_NSLOT,)),
        ],
    )
    def sc_kernel(in_hbm, emb_hbm, out_hbm, emb_v, io_v, lsem, ssem):
        wid = lax.axis_index("s") * _NC + lax.axis_index("c")
        base = wid * rows_per_w

        def load_descs(ci, slot):
            row0 = base + ci * _CHUNK
            ds = [pltpu.make_async_copy(emb_hbm.at[pl.ds(row0, _CHUNK)],
                                        emb_v.at[slot], lsem.at[slot])]
            for b in range(B):
                ds.append(pltpu.make_async_copy(in_hbm.at[b, pl.ds(row0, _CHUNK)],
                                                io_v.at[slot, b], lsem.at[slot]))
            return ds

        def store_descs(ci, slot):
            row0 = base + ci * _CHUNK
            return [pltpu.make_async_copy(io_v.at[slot, b],
                                          out_hbm.at[b, pl.ds(row0, _CHUNK)],
                                          ssem.at[slot])
                    for b in range(B)]

        def start_loads(ci, slot):
            for d in load_descs(ci, slot):
                d.start()

        def wait_loads(ci, slot):
            for d in load_descs(ci, slot):
                d.wait()

        def start_stores(ci, slot):
            for d in store_descs(ci, slot):
                d.start()

        def wait_stores(ci, slot):
            for d in store_descs(ci, slot):
                d.wait()

        start_loads(0, 0)

        @pl.loop(0, n_chunks)
        def _chunk(ci):
            slot = lax.rem(ci, _NSLOT)
            nxt = lax.rem(ci + 1, _NSLOT)

            @pl.when(ci + 1 < n_chunks)
            def _():
                @pl.when(ci >= _NSLOT - 1)
                def _():
                    wait_stores(ci - (_NSLOT - 1), nxt)
                start_loads(ci + 1, nxt)

            wait_loads(ci, slot)

            @pl.loop(0, _CHUNK)
            def _row(r):
                @pl.loop(0, vecs_per_row, unroll=_UNROLL)
                def _col(cv):
                    sl = pl.ds(cv * _L, _L)
                    e = emb_v[slot, r, sl]
                    for b in range(B):
                        plsc.addupdate(io_v.at[slot, b, r, sl], e)

            start_stores(ci, slot)

        for tail in range(n_chunks - (_NSLOT - 1), n_chunks):
            wait_stores(tail, tail % _NSLOT)

    return sc_kernel


def kernel(inputs, embeddings):
    B, S, D = inputs.shape
    pos = embeddings[:S]
    return _build_sc_kernel(B, S, D, inputs.dtype)(inputs, pos)


# speedup vs baseline: 3.4552x; 1.0347x over previous
import functools

import jax
import jax.numpy as jnp
from jax import lax
from jax.experimental import pallas as pl
from jax.experimental.pallas import tpu as pltpu
from jax.experimental.pallas import tpu_sc as plsc

_NC, _NS, _L = 2, 16, 16
_NW = _NC * _NS
_CHUNK = 8
_NSLOT = 3
_UNROLL = 8


@functools.cache
def _build_sc_kernel(B, S, D, dtype):
    rows_per_w = S // _NW
    n_chunks = rows_per_w // _CHUNK
    vecs_per_row = D // _L

    mesh = plsc.VectorSubcoreMesh(core_axis_name="c", subcore_axis_name="s")

    @functools.partial(
        pl.kernel,
        out_type=jax.ShapeDtypeStruct((B, S, D), dtype),
        mesh=mesh,
        scratch_types=[
            pltpu.VMEM((_NSLOT, _CHUNK, D), jnp.float32),
            pltpu.VMEM((_NSLOT, B, _CHUNK, D), jnp.float32),
            pltpu.SemaphoreType.DMA((_NSLOT,)),
            pltpu.SemaphoreType.DMA((_NSLOT,)),
        ],
    )
    def sc_kernel(in_hbm, emb_hbm, out_hbm, emb_v, io_v, lsem, ssem):
        wid = lax.axis_index("s") * _NC + lax.axis_index("c")
        base = wid * rows_per_w

        def load_descs(ci, slot):
            row0 = base + ci * _CHUNK
            ds = [pltpu.make_async_copy(emb_hbm.at[pl.ds(row0, _CHUNK)],
                                        emb_v.at[slot], lsem.at[slot])]
            for b in range(B):
                ds.append(pltpu.make_async_copy(in_hbm.at[b, pl.ds(row0, _CHUNK)],
                                                io_v.at[slot, b], lsem.at[slot]))
            return ds

        def store_descs(ci, slot):
            row0 = base + ci * _CHUNK
            return [pltpu.make_async_copy(io_v.at[slot, b],
                                          out_hbm.at[b, pl.ds(row0, _CHUNK)],
                                          ssem.at[slot])
                    for b in range(B)]

        def start_loads(ci, slot):
            for d in load_descs(ci, slot):
                d.start()

        def wait_loads(ci, slot):
            for d in load_descs(ci, slot):
                d.wait()

        def start_stores(ci, slot):
            for d in store_descs(ci, slot):
                d.start()

        def wait_stores(ci, slot):
            for d in store_descs(ci, slot):
                d.wait()

        start_loads(0, 0)

        @pl.loop(0, n_chunks)
        def _chunk(ci):
            slot = lax.rem(ci, _NSLOT)
            nxt = lax.rem(ci + 1, _NSLOT)

            @pl.when(ci + 1 < n_chunks)
            def _():
                @pl.when(ci >= _NSLOT - 1)
                def _():
                    wait_stores(ci - (_NSLOT - 1), nxt)
                start_loads(ci + 1, nxt)

            wait_loads(ci, slot)

            @pl.loop(0, _CHUNK)
            def _row(r):
                @plsc.parallel_loop(0, vecs_per_row, unroll=_UNROLL)
                def _col(cv):
                    sl = pl.ds(cv * _L, _L)
                    e = emb_v[slot, r, sl]
                    for b in range(B):
                        plsc.addupdate(io_v.at[slot, b, r, sl], e)

            start_stores(ci, slot)

        for tail in range(n_chunks - (_NSLOT - 1), n_chunks):
            wait_stores(tail, tail % _NSLOT)

    return sc_kernel


def kernel(inputs, embeddings):
    B, S, D = inputs.shape
    pos = embeddings[:S]
    return _build_sc_kernel(B, S, D, inputs.dtype)(inputs, pos)
